# Initial kernel scaffold; baseline (speedup 1.0000x reference)
import jax
import jax.numpy as jnp
from jax.experimental import pallas as pl


def kernel(x, edge_index, W_l0, b_l0, W_r0, W_l1, b_l1, W_r1):
    raise NotImplementedError("write your pallas kernel here")


# speedup vs baseline: 3.5877x; 3.5877x over previous
import jax
import jax.numpy as jnp
from jax import lax
from jax.experimental import pallas as pl
from jax.experimental.pallas import tpu as pltpu
from jax.experimental.pallas import tpu_sc as plsc

N_NODES = 10000
D = 128
E = 320000

NC = 2
NS = 16
NW = NC * NS

CH = 128
E_PER_W = E // NW
NCH = 80
E_PAD_W = NCH * CH
N_PAD = 10240
RPT = N_PAD // NS
ZC = RPT // CH
G = 16
NG = NCH // G

_mesh = plsc.VectorSubcoreMesh(core_axis_name="c", subcore_axis_name="s")


def _agg_body(y_hbm, srcs_hbm, dsts_hbm, agg_out, src_v, dst_v, rows_v,
              agg_sh, sem):
  c = lax.axis_index("c")
  s = lax.axis_index("s")
  wid = s * NC + c

  zero16 = jnp.zeros((16,), jnp.float32)

  def zrow(i, carry):
    for j in range(D // 16):
      rows_v[i, pl.ds(j * 16, 16)] = zero16
    return carry
  lax.fori_loop(0, CH, zrow, 0)
  for z in range(ZC):
    pltpu.sync_copy(rows_v, agg_sh.at[pl.ds(s * RPT + z * CH, CH)])

  plsc.subcore_barrier()

  def chunk(j, carry):
    pltpu.async_copy(y_hbm.at[src_v.at[j]], rows_v, sem).wait()
    pltpu.sync_copy(rows_v, agg_sh.at[dst_v.at[j]], add=True)
    return carry
  for g in range(NG):
    pltpu.sync_copy(srcs_hbm.at[wid, pl.ds(g * G, G)], src_v)
    pltpu.sync_copy(dsts_hbm.at[wid, pl.ds(g * G, G)], dst_v)
    lax.fori_loop(0, G, chunk, 0)

  plsc.subcore_barrier()

  sl = pl.ds(s * RPT, RPT)
  pltpu.sync_copy(agg_sh.at[sl], agg_out.at[c, sl])


_sc_agg = pl.kernel(
    _agg_body,
    out_type=jax.ShapeDtypeStruct((NC, N_PAD, D), jnp.float32),
    mesh=_mesh,
    scratch_types=[
        pltpu.VMEM((G, CH), jnp.int32),
        pltpu.VMEM((G, CH), jnp.int32),
        pltpu.VMEM((CH, D), jnp.float32),
        pltpu.VMEM_SHARED((N_PAD, D), jnp.float32),
        pltpu.SemaphoreType.DMA,
    ])


def _deg_body(dsts_hbm, deg_out, dst_v, ones_v, deg_sh):
  c = lax.axis_index("c")
  s = lax.axis_index("s")
  wid = s * NC + c

  zero16 = jnp.zeros((16,), jnp.float32)
  one16 = jnp.ones((16,), jnp.float32)

  def zrow(i, carry):
    for j in range(D // 16):
      ones_v[i, pl.ds(j * 16, 16)] = zero16
    return carry
  lax.fori_loop(0, CH, zrow, 0)
  for z in range(ZC):
    pltpu.sync_copy(ones_v, deg_sh.at[pl.ds(s * RPT + z * CH, CH)])

  def orow(i, carry):
    for j in range(D // 16):
      ones_v[i, pl.ds(j * 16, 16)] = one16
    return carry
  lax.fori_loop(0, CH, orow, 0)

  plsc.subcore_barrier()

  def chunk(j, carry):
    pltpu.sync_copy(ones_v, deg_sh.at[dst_v.at[j]], add=True)
    return carry
  for g in range(NG):
    pltpu.sync_copy(dsts_hbm.at[wid, pl.ds(g * G, G)], dst_v)
    lax.fori_loop(0, G, chunk, 0)

  plsc.subcore_barrier()

  sl = pl.ds(s * RPT, RPT)
  pltpu.sync_copy(deg_sh.at[sl], deg_out.at[c, sl])


_sc_degree = pl.kernel(
    _deg_body,
    out_type=jax.ShapeDtypeStruct((NC, N_PAD, D), jnp.float32),
    mesh=_mesh,
    scratch_types=[
        pltpu.VMEM((G, CH), jnp.int32),
        pltpu.VMEM((CH, D), jnp.float32),
        pltpu.VMEM_SHARED((N_PAD, D), jnp.float32),
    ])


_DOT_T = (((1,), (1,)), ((), ()))


def _proj_body(x_ref, wl_ref, wr_ref, b_ref, y_ref, r_ref):
  xx = x_ref[...]
  y_ref[...] = lax.dot_general(xx, wl_ref[...], _DOT_T,
                               preferred_element_type=jnp.float32)
  r_ref[...] = lax.dot_general(xx, wr_ref[...], _DOT_T,
                               preferred_element_type=jnp.float32) + b_ref[...]


def _mid_body(agg_ref, degp_ref, r0_ref, wl_ref, wr_ref, b_ref, y_ref, r_ref):
  deg = jnp.maximum(degp_ref[0] + degp_ref[1], 1.0)
  mean = (agg_ref[0] + agg_ref[1]) / deg
  h = jnp.maximum(mean + r0_ref[...], 0.0)
  y_ref[...] = lax.dot_general(h, wl_ref[...], _DOT_T,
                               preferred_element_type=jnp.float32)
  r_ref[...] = lax.dot_general(h, wr_ref[...], _DOT_T,
                               preferred_element_type=jnp.float32) + b_ref[...]


def _final_body(agg_ref, degp_ref, r1_ref, out_ref):
  deg = jnp.maximum(degp_ref[0] + degp_ref[1], 1.0)
  out_ref[...] = (agg_ref[0] + agg_ref[1]) / deg + r1_ref[...]


M_BLK = 1000
_GRID = N_NODES // M_BLK

_row_spec = pl.BlockSpec((M_BLK, D), lambda i: (i, 0))
_w_spec = pl.BlockSpec((D, D), lambda i: (0, 0))
_b_spec = pl.BlockSpec((1, D), lambda i: (0, 0))
_agg_spec = pl.BlockSpec((NC, M_BLK, D), lambda i: (0, i, 0))
_row_out = jax.ShapeDtypeStruct((N_NODES, D), jnp.float32)

_proj = pl.pallas_call(
    _proj_body, grid=(_GRID,),
    in_specs=[_row_spec, _w_spec, _w_spec, _b_spec],
    out_specs=[_row_spec, _row_spec],
    out_shape=[_row_out, _row_out])

_mid = pl.pallas_call(
    _mid_body, grid=(_GRID,),
    in_specs=[_agg_spec, _agg_spec, _row_spec, _w_spec, _w_spec, _b_spec],
    out_specs=[_row_spec, _row_spec],
    out_shape=[_row_out, _row_out])

_final = pl.pallas_call(
    _final_body, grid=(_GRID,),
    in_specs=[_agg_spec, _agg_spec, _row_spec],
    out_specs=_row_spec,
    out_shape=_row_out)


def kernel(x, edge_index, W_l0, b_l0, W_r0, W_l1, b_l1, W_r1):
  ei = edge_index.astype(jnp.int32).reshape(2, NW, E_PER_W)
  pad = ((0, 0), (0, E_PAD_W - E_PER_W))
  srcs = jnp.pad(ei[0], pad).reshape(NW, NCH, CH)
  dsts = jnp.pad(ei[1], pad, constant_values=N_NODES).reshape(NW, NCH, CH)

  degp = _sc_degree(dsts)
  y0, r0 = _proj(x, W_l0, W_r0, b_l0.reshape(1, D))
  agg0 = _sc_agg(y0, srcs, dsts)
  y1, r1 = _mid(agg0, degp, r0, W_l1, W_r1, b_l1.reshape(1, D))
  agg1 = _sc_agg(y1, srcs, dsts)
  return _final(agg1, degp, r1)
